# Initial kernel scaffold; baseline (speedup 1.0000x reference)
import jax
import jax.numpy as jnp
from jax.experimental import pallas as pl


def kernel(x, edge_index, pos_src, pos_dst, neg_src, neg_dst, Wself0, Wneigh0, b0, Wself1, Wneigh1, b1, Wself2, Wneigh2, b2, P0w, P0b, P1w, P1b, P2w, P2b):
    raise NotImplementedError("write your pallas kernel here")


# speedup vs baseline: 4.9926x; 4.9926x over previous
import jax
import jax.numpy as jnp
from jax import lax
from jax.experimental import pallas as pl
from jax.experimental.pallas import tpu as pltpu
from jax.experimental.pallas import tpu_sc as plsc

N = 10000
NP = 10240
E = 320000
H = 128
HH = H // 2
NPAIR = 16384

CHUNK = 125
EPS = E // 16
NCH = EPS // CHUNK
RPS = NP // 16


def _sc_agg(hA, hB, srcT, dstT, ones8, z64, z8):
    mesh = plsc.VectorSubcoreMesh(core_axis_name="c", subcore_axis_name="s")

    def body(hA_hbm, hB_hbm, src_hbm, dst_hbm, ones_hbm, z64_hbm, z8_hbm,
             out_agg, out_deg, idx_s, idx_d, rows, ones_v, acc, accd, sem):
        c = lax.axis_index("c")
        s = lax.axis_index("s")
        pltpu.sync_copy(z64_hbm, acc.at[pl.ds(s * RPS, RPS)])
        pltpu.sync_copy(ones_hbm, ones_v)
        pltpu.sync_copy(src_hbm.at[pl.ds(s * NCH, NCH)], idx_s)
        pltpu.sync_copy(dst_hbm.at[pl.ds(s * NCH, NCH)], idx_d)

        @pl.when(c == 0)
        def _():
            pltpu.sync_copy(z8_hbm, accd.at[pl.ds(s * RPS, RPS)])
        plsc.subcore_barrier()

        @pl.when(c == 0)
        def _():
            def step(j, carry):
                pltpu.async_copy(hA_hbm.at[idx_s.at[j]], rows, sem).wait()
                pltpu.sync_copy(rows, acc.at[idx_d.at[j]], add=True)
                pltpu.sync_copy(ones_v, accd.at[idx_d.at[j]], add=True)
                return carry
            lax.fori_loop(0, NCH, step, 0)

        @pl.when(c == 1)
        def _():
            def step(j, carry):
                pltpu.async_copy(hB_hbm.at[idx_s.at[j]], rows, sem).wait()
                pltpu.sync_copy(rows, acc.at[idx_d.at[j]], add=True)
                return carry
            lax.fori_loop(0, NCH, step, 0)

        plsc.subcore_barrier()
        pltpu.sync_copy(acc.at[pl.ds(s * RPS, RPS)],
                        out_agg.at[c, pl.ds(s * RPS, RPS)])

        @pl.when(c == 0)
        def _():
            pltpu.sync_copy(accd.at[pl.ds(s * RPS, RPS)],
                            out_deg.at[pl.ds(s * RPS, RPS)])

    fn = pl.kernel(
        body,
        out_type=(jax.ShapeDtypeStruct((2, NP, HH), jnp.float32),
                  jax.ShapeDtypeStruct((NP, 8), jnp.float32)),
        mesh=mesh,
        scratch_types=[
            pltpu.VMEM((NCH, CHUNK), jnp.int32),
            pltpu.VMEM((NCH, CHUNK), jnp.int32),
            pltpu.VMEM((CHUNK, HH), jnp.float32),
            pltpu.VMEM((CHUNK, 8), jnp.float32),
            pltpu.VMEM_SHARED((NP, HH), jnp.float32),
            pltpu.VMEM_SHARED((NP, 8), jnp.float32),
            pltpu.SemaphoreType.DMA,
        ],
        compiler_params=pltpu.CompilerParams(use_tc_tiling_on_sc=False),
    )
    return fn(hA, hB, srcT, dstT, ones8, z64, z8)


def _tc_layer(h2, part, deg8, Wself, Wneigh, b, relu, split_out):
    BN = 1024

    def body(h_ref, p_ref, d_ref, ws_ref, wn_ref, b_ref, o_ref):
        h = jnp.concatenate([h_ref[0], h_ref[1]], axis=1)
        deg = d_ref[:, 0:1]
        inv = 1.0 / jnp.maximum(deg, 1.0)
        mean = jnp.concatenate([p_ref[0], p_ref[1]], axis=1) * inv
        out = (jnp.dot(h, ws_ref[...], preferred_element_type=jnp.float32)
               + jnp.dot(mean, wn_ref[...], preferred_element_type=jnp.float32)
               + b_ref[...])
        if relu:
            out = jnp.maximum(out, 0.0)
        if split_out:
            o_ref[0] = out[:, :HH]
            o_ref[1] = out[:, HH:]
        else:
            o_ref[...] = out

    if split_out:
        out_shape = jax.ShapeDtypeStruct((2, NP, HH), jnp.float32)
        out_specs = pl.BlockSpec((2, BN, HH), lambda i: (0, i, 0))
    else:
        out_shape = jax.ShapeDtypeStruct((NP, H), jnp.float32)
        out_specs = pl.BlockSpec((BN, H), lambda i: (i, 0))

    return pl.pallas_call(
        body,
        grid=(NP // BN,),
        in_specs=[
            pl.BlockSpec((2, BN, HH), lambda i: (0, i, 0)),
            pl.BlockSpec((2, BN, HH), lambda i: (0, i, 0)),
            pl.BlockSpec((BN, 8), lambda i: (i, 0)),
            pl.BlockSpec((H, H), lambda i: (0, 0)),
            pl.BlockSpec((H, H), lambda i: (0, 0)),
            pl.BlockSpec((1, H), lambda i: (0, 0)),
        ],
        out_specs=out_specs,
        out_shape=out_shape,
    )(h2, part, deg8, Wself, Wneigh, b.reshape(1, H))


PG = 4 * NPAIR
GPW = PG // 32
GCH = GPW // 128


def _sc_gather(h, idxT):
    mesh = plsc.VectorSubcoreMesh(core_axis_name="c", subcore_axis_name="s")

    def body(h_hbm, idx_hbm, out_hbm, idx_v, rows, sem):
        c = lax.axis_index("c")
        s = lax.axis_index("s")
        wid = s * 2 + c
        pltpu.sync_copy(idx_hbm.at[pl.ds(wid * GCH, GCH)], idx_v)

        def step(j, carry):
            pltpu.async_copy(h_hbm.at[idx_v.at[j]], rows, sem).wait()
            pltpu.sync_copy(rows, out_hbm.at[pl.ds(wid * GPW + j * 128, 128)])
            return carry

        lax.fori_loop(0, GCH, step, 0)

    fn = pl.kernel(
        body,
        out_type=jax.ShapeDtypeStruct((PG, H), jnp.float32),
        mesh=mesh,
        scratch_types=[
            pltpu.VMEM((GCH, 128), jnp.int32),
            pltpu.VMEM((128, H), jnp.float32),
            pltpu.SemaphoreType.DMA,
        ],
    )
    return fn(h, idxT)


def _tc_pred(hs, hd, P0w, P0b, P1w, P1b, P2w, P2b):
    BP = 2048
    M = hs.shape[0]

    def body(a_ref, b_ref, w0, c0, w1, c1, w2, c2, o_ref):
        z = a_ref[...] * b_ref[...]
        z = jnp.maximum(jnp.dot(z, w0[...], preferred_element_type=jnp.float32) + c0[...], 0.0)
        z = jnp.maximum(jnp.dot(z, w1[...], preferred_element_type=jnp.float32) + c1[...], 0.0)
        logit = jnp.dot(z, w2[...], preferred_element_type=jnp.float32) + c2[...]
        d = logit[:, 1:2] - logit[:, 0:1]
        o_ref[...] = 1.0 / (1.0 + jnp.exp(-d))

    return pl.pallas_call(
        body,
        grid=(M // BP,),
        in_specs=[
            pl.BlockSpec((BP, H), lambda i: (i, 0)),
            pl.BlockSpec((BP, H), lambda i: (i, 0)),
            pl.BlockSpec((H, H), lambda i: (0, 0)),
            pl.BlockSpec((1, H), lambda i: (0, 0)),
            pl.BlockSpec((H, H), lambda i: (0, 0)),
            pl.BlockSpec((1, H), lambda i: (0, 0)),
            pl.BlockSpec((H, 2), lambda i: (0, 0)),
            pl.BlockSpec((1, 2), lambda i: (0, 0)),
        ],
        out_specs=pl.BlockSpec((BP, 1), lambda i: (i, 0)),
        out_shape=jax.ShapeDtypeStruct((M, 1), jnp.float32),
    )(hs, hd, P0w, P0b.reshape(1, H), P1w, P1b.reshape(1, H), P2w,
      P2b.reshape(1, 2))


def kernel(x, edge_index, pos_src, pos_dst, neg_src, neg_dst,
           Wself0, Wneigh0, b0, Wself1, Wneigh1, b1, Wself2, Wneigh2, b2,
           P0w, P0b, P1w, P1b, P2w, P2b):
    srcT = edge_index[0].reshape(E // CHUNK, CHUNK)
    dstT = edge_index[1].reshape(E // CHUNK, CHUNK)
    ones8 = jnp.ones((CHUNK, 8), jnp.float32)
    z64 = jnp.zeros((RPS, HH), jnp.float32)
    z8 = jnp.zeros((RPS, 8), jnp.float32)

    xp = jnp.pad(x, ((0, NP - N), (0, 0)))
    h2 = jnp.stack([xp[:, :HH], xp[:, HH:]])
    part, deg8 = _sc_agg(h2[0], h2[1], srcT, dstT, ones8, z64, z8)
    h2 = _tc_layer(h2, part, deg8, Wself0, Wneigh0, b0, True, True)
    part, _ = _sc_agg(h2[0], h2[1], srcT, dstT, ones8, z64, z8)
    h2 = _tc_layer(h2, part, deg8, Wself1, Wneigh1, b1, True, True)
    part, _ = _sc_agg(h2[0], h2[1], srcT, dstT, ones8, z64, z8)
    h = _tc_layer(h2, part, deg8, Wself2, Wneigh2, b2, False, False)

    idxT = jnp.concatenate([pos_src, neg_src, pos_dst, neg_dst]).reshape(
        PG // 128, 128)
    rows = _sc_gather(h, idxT)
    hs = rows[:2 * NPAIR]
    hd = rows[2 * NPAIR:]
    scores = _tc_pred(hs, hd, P0w, P0b, P1w, P1b, P2w, P2b)[:, 0]
    return scores[:NPAIR], scores[NPAIR:]


# speedup vs baseline: 6.3108x; 1.2640x over previous
import jax
import jax.numpy as jnp
from jax import lax
from jax.experimental import pallas as pl
from jax.experimental.pallas import tpu as pltpu
from jax.experimental.pallas import tpu_sc as plsc

N = 10000
NP = 10240
E = 320000
H = 128
HH = H // 2
NPAIR = 16384

CHUNK = 125
EPS = E // 16
NCH = EPS // CHUNK
RPS = NP // 16


def _sc_agg(hA, hB, srcT, dstT, ones8, z64, z8, with_deg):
    mesh = plsc.VectorSubcoreMesh(core_axis_name="c", subcore_axis_name="s")

    def body(*refs):
        if with_deg:
            (hA_hbm, hB_hbm, src_hbm, dst_hbm, ones_hbm, z64_hbm, z8_hbm,
             out_agg, out_deg, idx_s, idx_d, rows0, rows1, ones_v, acc, accd,
             sem0, sem1) = refs
        else:
            (hA_hbm, hB_hbm, src_hbm, dst_hbm, z64_hbm,
             out_agg, idx_s, idx_d, rows0, rows1, acc, sem0, sem1) = refs
        c = lax.axis_index("c")
        s = lax.axis_index("s")
        pltpu.sync_copy(z64_hbm, acc.at[pl.ds(s * RPS, RPS)])
        pltpu.sync_copy(src_hbm.at[pl.ds(s * NCH, NCH)], idx_s)
        pltpu.sync_copy(dst_hbm.at[pl.ds(s * NCH, NCH)], idx_d)
        if with_deg:
            pltpu.sync_copy(ones_hbm, ones_v)

            @pl.when(c == 0)
            def _():
                pltpu.sync_copy(z8_hbm, accd.at[pl.ds(s * RPS, RPS)])
        plsc.subcore_barrier()

        def run(h_ref, do_deg):
            pltpu.async_copy(h_ref.at[idx_s.at[0]], rows0, sem0)

            def step(i, carry):
                j0 = 2 * i
                j1 = 2 * i + 1
                pltpu.make_async_copy(
                    h_ref.at[idx_s.at[j0]], rows0, sem0).wait()
                pltpu.async_copy(h_ref.at[idx_s.at[j1]], rows1, sem1)
                pltpu.sync_copy(rows0, acc.at[idx_d.at[j0]], add=True)
                if do_deg:
                    pltpu.sync_copy(ones_v, accd.at[idx_d.at[j0]], add=True)
                pltpu.make_async_copy(
                    h_ref.at[idx_s.at[j1]], rows1, sem1).wait()
                pltpu.async_copy(
                    h_ref.at[idx_s.at[jnp.minimum(j0 + 2, NCH - 1)]],
                    rows0, sem0)
                pltpu.sync_copy(rows1, acc.at[idx_d.at[j1]], add=True)
                if do_deg:
                    pltpu.sync_copy(ones_v, accd.at[idx_d.at[j1]], add=True)
                return carry

            lax.fori_loop(0, NCH // 2, step, 0)
            pltpu.make_async_copy(
                h_ref.at[idx_s.at[NCH - 1]], rows0, sem0).wait()

        @pl.when(c == 0)
        def _():
            run(hA_hbm, with_deg)

        @pl.when(c == 1)
        def _():
            run(hB_hbm, False)

        plsc.subcore_barrier()
        pltpu.sync_copy(acc.at[pl.ds(s * RPS, RPS)],
                        out_agg.at[c, pl.ds(s * RPS, RPS)])

        if with_deg:
            @pl.when(c == 0)
            def _():
                pltpu.sync_copy(accd.at[pl.ds(s * RPS, RPS)],
                                out_deg.at[pl.ds(s * RPS, RPS)])

    out_type = [jax.ShapeDtypeStruct((2, NP, HH), jnp.float32)]
    scratch = [
        pltpu.VMEM((NCH, CHUNK), jnp.int32),
        pltpu.VMEM((NCH, CHUNK), jnp.int32),
        pltpu.VMEM((CHUNK, HH), jnp.float32),
        pltpu.VMEM((CHUNK, HH), jnp.float32),
    ]
    if with_deg:
        out_type.append(jax.ShapeDtypeStruct((NP, 8), jnp.float32))
        scratch.append(pltpu.VMEM((CHUNK, 8), jnp.float32))
    scratch.append(pltpu.VMEM_SHARED((NP, HH), jnp.float32))
    if with_deg:
        scratch.append(pltpu.VMEM_SHARED((NP, 8), jnp.float32))
    scratch += [pltpu.SemaphoreType.DMA, pltpu.SemaphoreType.DMA]

    fn = pl.kernel(
        body,
        out_type=tuple(out_type),
        mesh=mesh,
        scratch_types=scratch,
        compiler_params=pltpu.CompilerParams(use_tc_tiling_on_sc=False),
    )
    if with_deg:
        return fn(hA, hB, srcT, dstT, ones8, z64, z8)
    return fn(hA, hB, srcT, dstT, z64)[0]


def _tc_layer(h2, part, deg8, Wself, Wneigh, b, relu, split_out):
    BN = 1024

    def body(h_ref, p_ref, d_ref, ws_ref, wn_ref, b_ref, o_ref):
        h = jnp.concatenate([h_ref[0], h_ref[1]], axis=1)
        deg = d_ref[:, 0:1]
        inv = 1.0 / jnp.maximum(deg, 1.0)
        mean = jnp.concatenate([p_ref[0], p_ref[1]], axis=1) * inv
        out = (jnp.dot(h, ws_ref[...], preferred_element_type=jnp.float32)
               + jnp.dot(mean, wn_ref[...], preferred_element_type=jnp.float32)
               + b_ref[...])
        if relu:
            out = jnp.maximum(out, 0.0)
        if split_out:
            o_ref[0] = out[:, :HH]
            o_ref[1] = out[:, HH:]
        else:
            o_ref[...] = out

    if split_out:
        out_shape = jax.ShapeDtypeStruct((2, NP, HH), jnp.float32)
        out_specs = pl.BlockSpec((2, BN, HH), lambda i: (0, i, 0))
    else:
        out_shape = jax.ShapeDtypeStruct((NP, H), jnp.float32)
        out_specs = pl.BlockSpec((BN, H), lambda i: (i, 0))

    return pl.pallas_call(
        body,
        grid=(NP // BN,),
        in_specs=[
            pl.BlockSpec((2, BN, HH), lambda i: (0, i, 0)),
            pl.BlockSpec((2, BN, HH), lambda i: (0, i, 0)),
            pl.BlockSpec((BN, 8), lambda i: (i, 0)),
            pl.BlockSpec((H, H), lambda i: (0, 0)),
            pl.BlockSpec((H, H), lambda i: (0, 0)),
            pl.BlockSpec((1, H), lambda i: (0, 0)),
        ],
        out_specs=out_specs,
        out_shape=out_shape,
    )(h2, part, deg8, Wself, Wneigh, b.reshape(1, H))


PG = 4 * NPAIR
GPW = PG // 32
GCH = GPW // 128


def _sc_gather(h, idxT):
    mesh = plsc.VectorSubcoreMesh(core_axis_name="c", subcore_axis_name="s")

    def body(h_hbm, idx_hbm, out_hbm, idx_v, rows, sem):
        c = lax.axis_index("c")
        s = lax.axis_index("s")
        wid = s * 2 + c
        pltpu.sync_copy(idx_hbm.at[pl.ds(wid * GCH, GCH)], idx_v)

        def step(j, carry):
            pltpu.async_copy(h_hbm.at[idx_v.at[j]], rows, sem).wait()
            pltpu.sync_copy(rows, out_hbm.at[pl.ds(wid * GPW + j * 128, 128)])
            return carry

        lax.fori_loop(0, GCH, step, 0)

    fn = pl.kernel(
        body,
        out_type=jax.ShapeDtypeStruct((PG, H), jnp.float32),
        mesh=mesh,
        scratch_types=[
            pltpu.VMEM((GCH, 128), jnp.int32),
            pltpu.VMEM((128, H), jnp.float32),
            pltpu.SemaphoreType.DMA,
        ],
    )
    return fn(h, idxT)


def _tc_pred(hs, hd, P0w, P0b, P1w, P1b, P2w, P2b):
    BP = 2048
    M = hs.shape[0]

    def body(a_ref, b_ref, w0, c0, w1, c1, w2, c2, o_ref):
        z = a_ref[...] * b_ref[...]
        z = jnp.maximum(jnp.dot(z, w0[...], preferred_element_type=jnp.float32) + c0[...], 0.0)
        z = jnp.maximum(jnp.dot(z, w1[...], preferred_element_type=jnp.float32) + c1[...], 0.0)
        logit = jnp.dot(z, w2[...], preferred_element_type=jnp.float32) + c2[...]
        d = logit[:, 1:2] - logit[:, 0:1]
        o_ref[...] = 1.0 / (1.0 + jnp.exp(-d))

    return pl.pallas_call(
        body,
        grid=(M // BP,),
        in_specs=[
            pl.BlockSpec((BP, H), lambda i: (i, 0)),
            pl.BlockSpec((BP, H), lambda i: (i, 0)),
            pl.BlockSpec((H, H), lambda i: (0, 0)),
            pl.BlockSpec((1, H), lambda i: (0, 0)),
            pl.BlockSpec((H, H), lambda i: (0, 0)),
            pl.BlockSpec((1, H), lambda i: (0, 0)),
            pl.BlockSpec((H, 2), lambda i: (0, 0)),
            pl.BlockSpec((1, 2), lambda i: (0, 0)),
        ],
        out_specs=pl.BlockSpec((BP, 1), lambda i: (i, 0)),
        out_shape=jax.ShapeDtypeStruct((M, 1), jnp.float32),
    )(hs, hd, P0w, P0b.reshape(1, H), P1w, P1b.reshape(1, H), P2w,
      P2b.reshape(1, 2))


def kernel(x, edge_index, pos_src, pos_dst, neg_src, neg_dst,
           Wself0, Wneigh0, b0, Wself1, Wneigh1, b1, Wself2, Wneigh2, b2,
           P0w, P0b, P1w, P1b, P2w, P2b):
    srcT = edge_index[0].reshape(E // CHUNK, CHUNK)
    dstT = edge_index[1].reshape(E // CHUNK, CHUNK)
    ones8 = jnp.ones((CHUNK, 8), jnp.float32)
    z64 = jnp.zeros((RPS, HH), jnp.float32)
    z8 = jnp.zeros((RPS, 8), jnp.float32)

    xp = jnp.pad(x, ((0, NP - N), (0, 0)))
    h2 = jnp.stack([xp[:, :HH], xp[:, HH:]])
    part, deg8 = _sc_agg(h2[0], h2[1], srcT, dstT, ones8, z64, z8, True)
    h2 = _tc_layer(h2, part, deg8, Wself0, Wneigh0, b0, True, True)
    part = _sc_agg(h2[0], h2[1], srcT, dstT, ones8, z64, z8, False)
    h2 = _tc_layer(h2, part, deg8, Wself1, Wneigh1, b1, True, True)
    part = _sc_agg(h2[0], h2[1], srcT, dstT, ones8, z64, z8, False)
    h = _tc_layer(h2, part, deg8, Wself2, Wneigh2, b2, False, False)

    idxT = jnp.concatenate([pos_src, neg_src, pos_dst, neg_dst]).reshape(
        PG // 128, 128)
    rows = _sc_gather(h, idxT)
    hs = rows[:2 * NPAIR]
    hd = rows[2 * NPAIR:]
    scores = _tc_pred(hs, hd, P0w, P0b, P1w, P1b, P2w, P2b)[:, 0]
    return scores[:NPAIR], scores[NPAIR:]


# speedup vs baseline: 8.3790x; 1.3277x over previous
import jax
import jax.numpy as jnp
from jax import lax
from jax.experimental import pallas as pl
from jax.experimental.pallas import tpu as pltpu
from jax.experimental.pallas import tpu_sc as plsc

N = 10000
NP = 10240
E = 320000
H = 128
HH = H // 2
NPAIR = 16384

CHUNK = 125
EPS = E // 16
NCH = EPS // CHUNK
RPS = NP // 16


def _sc_agg(hA, hB, srcT, dstT, ones8, z64, z8, with_deg):
    mesh = plsc.VectorSubcoreMesh(core_axis_name="c", subcore_axis_name="s")

    def body(*refs):
        if with_deg:
            (hA_hbm, hB_hbm, src_hbm, dst_hbm, ones_hbm, z64_hbm, z8_hbm,
             out_agg, out_deg, idx_s, idx_d, rows0, rows1, rows2, rows3,
             ones_v, acc, accd, sem_g, sem_s) = refs
        else:
            (hA_hbm, hB_hbm, src_hbm, dst_hbm, z64_hbm,
             out_agg, idx_s, idx_d, rows0, rows1, rows2, rows3, acc,
             sem_g, sem_s) = refs
        c = lax.axis_index("c")
        s = lax.axis_index("s")
        pltpu.sync_copy(z64_hbm, acc.at[pl.ds(s * RPS, RPS)])
        pltpu.sync_copy(src_hbm.at[pl.ds(s * NCH, NCH)], idx_s)
        pltpu.sync_copy(dst_hbm.at[pl.ds(s * NCH, NCH)], idx_d)
        if with_deg:
            pltpu.sync_copy(ones_hbm, ones_v)

            @pl.when(c == 0)
            def _():
                pltpu.sync_copy(z8_hbm, accd.at[pl.ds(s * RPS, RPS)])
        plsc.subcore_barrier()

        def run(h_ref, do_deg):
            bufs = (rows0, rows1, rows2, rows3)

            def wait_g(j, k):
                pltpu.make_async_copy(
                    h_ref.at[idx_s.at[j]], bufs[k], sem_g).wait()

            def wait_s(j, k):
                pltpu.make_async_copy(
                    bufs[k], acc.at[idx_d.at[j]], sem_s).wait()

            def start_g(j, k):
                pltpu.async_copy(h_ref.at[idx_s.at[j]], bufs[k], sem_g)

            def start_s(j, k):
                pltpu.async_copy(bufs[k], acc.at[idx_d.at[j]], sem_s,
                                 add=True)

            def stage(j, k, do_wait_s, do_start_g):
                if do_wait_s:
                    wait_s(j - 2, (k + 2) % 4)
                wait_g(j, k)
                start_s(j, k)
                if do_deg:
                    pltpu.sync_copy(ones_v, accd.at[idx_d.at[j]], add=True)
                if do_start_g:
                    start_g(j + 2, (k + 2) % 4)

            start_g(0, 0)
            start_g(1, 1)
            stage(0, 0, False, True)
            stage(1, 1, False, True)
            stage(2, 2, True, True)
            stage(3, 3, True, True)

            def group(g, carry):
                j = 4 * g
                stage(j + 0, 0, True, True)
                stage(j + 1, 1, True, True)
                stage(j + 2, 2, True, True)
                stage(j + 3, 3, True, True)
                return carry

            lax.fori_loop(1, NCH // 4 - 1, group, 0)
            j = NCH - 4
            stage(j + 0, 0, True, True)
            stage(j + 1, 1, True, True)
            stage(j + 2, 2, True, False)
            stage(j + 3, 3, True, False)
            wait_s(NCH - 2, 2)
            wait_s(NCH - 1, 3)

        @pl.when(c == 0)
        def _():
            run(hA_hbm, with_deg)

        @pl.when(c == 1)
        def _():
            run(hB_hbm, False)

        plsc.subcore_barrier()
        pltpu.sync_copy(acc.at[pl.ds(s * RPS, RPS)],
                        out_agg.at[c, pl.ds(s * RPS, RPS)])

        if with_deg:
            @pl.when(c == 0)
            def _():
                pltpu.sync_copy(accd.at[pl.ds(s * RPS, RPS)],
                                out_deg.at[pl.ds(s * RPS, RPS)])

    out_type = [jax.ShapeDtypeStruct((2, NP, HH), jnp.float32)]
    scratch = [
        pltpu.VMEM((NCH, CHUNK), jnp.int32),
        pltpu.VMEM((NCH, CHUNK), jnp.int32),
        pltpu.VMEM((CHUNK, HH), jnp.float32),
        pltpu.VMEM((CHUNK, HH), jnp.float32),
        pltpu.VMEM((CHUNK, HH), jnp.float32),
        pltpu.VMEM((CHUNK, HH), jnp.float32),
    ]
    if with_deg:
        out_type.append(jax.ShapeDtypeStruct((NP, 8), jnp.float32))
        scratch.append(pltpu.VMEM((CHUNK, 8), jnp.float32))
    scratch.append(pltpu.VMEM_SHARED((NP, HH), jnp.float32))
    if with_deg:
        scratch.append(pltpu.VMEM_SHARED((NP, 8), jnp.float32))
    scratch += [pltpu.SemaphoreType.DMA, pltpu.SemaphoreType.DMA]

    fn = pl.kernel(
        body,
        out_type=tuple(out_type),
        mesh=mesh,
        scratch_types=scratch,
        compiler_params=pltpu.CompilerParams(use_tc_tiling_on_sc=False),
    )
    if with_deg:
        return fn(hA, hB, srcT, dstT, ones8, z64, z8)
    return fn(hA, hB, srcT, dstT, z64)[0]


def _tc_layer(h2, part, deg8, Wself, Wneigh, b, relu, split_out):
    BN = 1024

    def body(h_ref, p_ref, d_ref, ws_ref, wn_ref, b_ref, o_ref):
        h = jnp.concatenate([h_ref[0], h_ref[1]], axis=1)
        deg = d_ref[:, 0:1]
        inv = 1.0 / jnp.maximum(deg, 1.0)
        mean = jnp.concatenate([p_ref[0], p_ref[1]], axis=1) * inv
        out = (jnp.dot(h, ws_ref[...], preferred_element_type=jnp.float32)
               + jnp.dot(mean, wn_ref[...], preferred_element_type=jnp.float32)
               + b_ref[...])
        if relu:
            out = jnp.maximum(out, 0.0)
        if split_out:
            o_ref[0] = out[:, :HH]
            o_ref[1] = out[:, HH:]
        else:
            o_ref[...] = out

    if split_out:
        out_shape = jax.ShapeDtypeStruct((2, NP, HH), jnp.float32)
        out_specs = pl.BlockSpec((2, BN, HH), lambda i: (0, i, 0))
    else:
        out_shape = jax.ShapeDtypeStruct((NP, H), jnp.float32)
        out_specs = pl.BlockSpec((BN, H), lambda i: (i, 0))

    return pl.pallas_call(
        body,
        grid=(NP // BN,),
        in_specs=[
            pl.BlockSpec((2, BN, HH), lambda i: (0, i, 0)),
            pl.BlockSpec((2, BN, HH), lambda i: (0, i, 0)),
            pl.BlockSpec((BN, 8), lambda i: (i, 0)),
            pl.BlockSpec((H, H), lambda i: (0, 0)),
            pl.BlockSpec((H, H), lambda i: (0, 0)),
            pl.BlockSpec((1, H), lambda i: (0, 0)),
        ],
        out_specs=out_specs,
        out_shape=out_shape,
    )(h2, part, deg8, Wself, Wneigh, b.reshape(1, H))


PG = 4 * NPAIR
GPW = PG // 32
GCH = GPW // 128


def _sc_gather(h, idxT):
    mesh = plsc.VectorSubcoreMesh(core_axis_name="c", subcore_axis_name="s")

    def body(h_hbm, idx_hbm, out_hbm, idx_v,
             rows0, rows1, rows2, rows3, sem_g, sem_w):
        c = lax.axis_index("c")
        s = lax.axis_index("s")
        wid = s * 2 + c
        pltpu.sync_copy(idx_hbm.at[pl.ds(wid * GCH, GCH)], idx_v)
        bufs = (rows0, rows1, rows2, rows3)

        def out_at(j):
            return out_hbm.at[pl.ds(wid * GPW + j * 128, 128)]

        def start_g(j, k):
            pltpu.async_copy(h_hbm.at[idx_v.at[j]], bufs[k], sem_g)

        def stage(j, k, do_wait_w, do_start_g):
            if do_wait_w:
                pltpu.make_async_copy(
                    bufs[(k + 2) % 4], out_at(j - 2), sem_w).wait()
            pltpu.make_async_copy(h_hbm.at[idx_v.at[j]], bufs[k], sem_g).wait()
            pltpu.async_copy(bufs[k], out_at(j), sem_w)
            if do_start_g:
                start_g(j + 2, (k + 2) % 4)

        start_g(0, 0)
        start_g(1, 1)
        stage(0, 0, False, True)
        stage(1, 1, False, True)
        stage(2, 2, True, True)
        stage(3, 3, True, True)

        def group(g, carry):
            j = 4 * g
            stage(j + 0, 0, True, True)
            stage(j + 1, 1, True, True)
            stage(j + 2, 2, True, True)
            stage(j + 3, 3, True, True)
            return carry

        lax.fori_loop(1, GCH // 4 - 1, group, 0)
        j = GCH - 4
        stage(j + 0, 0, True, True)
        stage(j + 1, 1, True, True)
        stage(j + 2, 2, True, False)
        stage(j + 3, 3, True, False)
        pltpu.make_async_copy(bufs[2], out_at(GCH - 2), sem_w).wait()
        pltpu.make_async_copy(bufs[3], out_at(GCH - 1), sem_w).wait()

    fn = pl.kernel(
        body,
        out_type=jax.ShapeDtypeStruct((PG, H), jnp.float32),
        mesh=mesh,
        scratch_types=[
            pltpu.VMEM((GCH, 128), jnp.int32),
            pltpu.VMEM((128, H), jnp.float32),
            pltpu.VMEM((128, H), jnp.float32),
            pltpu.VMEM((128, H), jnp.float32),
            pltpu.VMEM((128, H), jnp.float32),
            pltpu.SemaphoreType.DMA,
            pltpu.SemaphoreType.DMA,
        ],
    )
    return fn(h, idxT)


def _tc_pred(hs, hd, P0w, P0b, P1w, P1b, P2w, P2b):
    BP = 2048
    M = hs.shape[0]

    def body(a_ref, b_ref, w0, c0, w1, c1, w2, c2, o_ref):
        z = a_ref[...] * b_ref[...]
        z = jnp.maximum(jnp.dot(z, w0[...], preferred_element_type=jnp.float32) + c0[...], 0.0)
        z = jnp.maximum(jnp.dot(z, w1[...], preferred_element_type=jnp.float32) + c1[...], 0.0)
        logit = jnp.dot(z, w2[...], preferred_element_type=jnp.float32) + c2[...]
        d = logit[:, 1:2] - logit[:, 0:1]
        o_ref[...] = 1.0 / (1.0 + jnp.exp(-d))

    return pl.pallas_call(
        body,
        grid=(M // BP,),
        in_specs=[
            pl.BlockSpec((BP, H), lambda i: (i, 0)),
            pl.BlockSpec((BP, H), lambda i: (i, 0)),
            pl.BlockSpec((H, H), lambda i: (0, 0)),
            pl.BlockSpec((1, H), lambda i: (0, 0)),
            pl.BlockSpec((H, H), lambda i: (0, 0)),
            pl.BlockSpec((1, H), lambda i: (0, 0)),
            pl.BlockSpec((H, 2), lambda i: (0, 0)),
            pl.BlockSpec((1, 2), lambda i: (0, 0)),
        ],
        out_specs=pl.BlockSpec((BP, 1), lambda i: (i, 0)),
        out_shape=jax.ShapeDtypeStruct((M, 1), jnp.float32),
    )(hs, hd, P0w, P0b.reshape(1, H), P1w, P1b.reshape(1, H), P2w,
      P2b.reshape(1, 2))


def kernel(x, edge_index, pos_src, pos_dst, neg_src, neg_dst,
           Wself0, Wneigh0, b0, Wself1, Wneigh1, b1, Wself2, Wneigh2, b2,
           P0w, P0b, P1w, P1b, P2w, P2b):
    srcT = edge_index[0].reshape(E // CHUNK, CHUNK)
    dstT = edge_index[1].reshape(E // CHUNK, CHUNK)
    ones8 = jnp.ones((CHUNK, 8), jnp.float32)
    z64 = jnp.zeros((RPS, HH), jnp.float32)
    z8 = jnp.zeros((RPS, 8), jnp.float32)

    xp = jnp.pad(x, ((0, NP - N), (0, 0)))
    h2 = jnp.stack([xp[:, :HH], xp[:, HH:]])
    part, deg8 = _sc_agg(h2[0], h2[1], srcT, dstT, ones8, z64, z8, True)
    h2 = _tc_layer(h2, part, deg8, Wself0, Wneigh0, b0, True, True)
    part = _sc_agg(h2[0], h2[1], srcT, dstT, ones8, z64, z8, False)
    h2 = _tc_layer(h2, part, deg8, Wself1, Wneigh1, b1, True, True)
    part = _sc_agg(h2[0], h2[1], srcT, dstT, ones8, z64, z8, False)
    h = _tc_layer(h2, part, deg8, Wself2, Wneigh2, b2, False, False)

    idxT = jnp.concatenate([pos_src, neg_src, pos_dst, neg_dst]).reshape(
        PG // 128, 128)
    rows = _sc_gather(h, idxT)
    hs = rows[:2 * NPAIR]
    hd = rows[2 * NPAIR:]
    scores = _tc_pred(hs, hd, P0w, P0b, P1w, P1b, P2w, P2b)[:, 0]
    return scores[:NPAIR], scores[NPAIR:]


# speedup vs baseline: 9.4031x; 1.1222x over previous
import jax
import jax.numpy as jnp
from jax import lax
from jax.experimental import pallas as pl
from jax.experimental.pallas import tpu as pltpu
from jax.experimental.pallas import tpu_sc as plsc

N = 10000
NP = 10240
E = 320000
H = 128
HH = H // 2
NPAIR = 16384

CHUNK = 125
EPS = E // 16
NCH = EPS // CHUNK
RPS = NP // 16


def _sc_agg(h2, ei3, ones8, z64, z8, with_deg):
    mesh = plsc.VectorSubcoreMesh(core_axis_name="c", subcore_axis_name="s")

    def body(*refs):
        if with_deg:
            (h2_hbm, ei_hbm, ones_hbm, z64_hbm, z8_hbm,
             out_agg, out_deg, idx_s, idx_d, rows0, rows1, rows2, rows3,
             ones_v, acc, accd, sem_g, sem_s) = refs
        else:
            (h2_hbm, ei_hbm, z64_hbm,
             out_agg, idx_s, idx_d, rows0, rows1, rows2, rows3, acc,
             sem_g, sem_s) = refs
        c = lax.axis_index("c")
        s = lax.axis_index("s")
        pltpu.sync_copy(z64_hbm, acc.at[pl.ds(s * RPS, RPS)])
        pltpu.sync_copy(ei_hbm.at[0, pl.ds(s * NCH, NCH)], idx_s)
        pltpu.sync_copy(ei_hbm.at[1, pl.ds(s * NCH, NCH)], idx_d)
        if with_deg:
            pltpu.sync_copy(ones_hbm, ones_v)

            @pl.when(c == 0)
            def _():
                pltpu.sync_copy(z8_hbm, accd.at[pl.ds(s * RPS, RPS)])
        plsc.subcore_barrier()

        def run(h_ref, do_deg):
            bufs = (rows0, rows1, rows2, rows3)

            def wait_g(j, k):
                pltpu.make_async_copy(
                    h_ref.at[idx_s.at[j]], bufs[k], sem_g).wait()

            def wait_s(j, k):
                pltpu.make_async_copy(
                    bufs[k], acc.at[idx_d.at[j]], sem_s).wait()

            def start_g(j, k):
                pltpu.async_copy(h_ref.at[idx_s.at[j]], bufs[k], sem_g)

            def start_s(j, k):
                pltpu.async_copy(bufs[k], acc.at[idx_d.at[j]], sem_s,
                                 add=True)

            def stage(j, k, do_wait_s, do_start_g):
                if do_wait_s:
                    wait_s(j - 2, (k + 2) % 4)
                wait_g(j, k)
                start_s(j, k)
                if do_deg:
                    pltpu.sync_copy(ones_v, accd.at[idx_d.at[j]], add=True)
                if do_start_g:
                    start_g(j + 2, (k + 2) % 4)

            start_g(0, 0)
            start_g(1, 1)
            stage(0, 0, False, True)
            stage(1, 1, False, True)
            stage(2, 2, True, True)
            stage(3, 3, True, True)

            def group(g, carry):
                j = 4 * g
                stage(j + 0, 0, True, True)
                stage(j + 1, 1, True, True)
                stage(j + 2, 2, True, True)
                stage(j + 3, 3, True, True)
                return carry

            lax.fori_loop(1, NCH // 4 - 1, group, 0)
            j = NCH - 4
            stage(j + 0, 0, True, True)
            stage(j + 1, 1, True, True)
            stage(j + 2, 2, True, False)
            stage(j + 3, 3, True, False)
            wait_s(NCH - 2, 2)
            wait_s(NCH - 1, 3)

        @pl.when(c == 0)
        def _():
            run(h2_hbm.at[0], with_deg)

        @pl.when(c == 1)
        def _():
            run(h2_hbm.at[1], False)

        plsc.subcore_barrier()
        pltpu.sync_copy(acc.at[pl.ds(s * RPS, RPS)],
                        out_agg.at[c, pl.ds(s * RPS, RPS)])

        if with_deg:
            @pl.when(c == 0)
            def _():
                pltpu.sync_copy(accd.at[pl.ds(s * RPS, RPS)],
                                out_deg.at[pl.ds(s * RPS, RPS)])

    out_type = [jax.ShapeDtypeStruct((2, NP, HH), jnp.float32)]
    scratch = [
        pltpu.VMEM((NCH, CHUNK), jnp.int32),
        pltpu.VMEM((NCH, CHUNK), jnp.int32),
        pltpu.VMEM((CHUNK, HH), jnp.float32),
        pltpu.VMEM((CHUNK, HH), jnp.float32),
        pltpu.VMEM((CHUNK, HH), jnp.float32),
        pltpu.VMEM((CHUNK, HH), jnp.float32),
    ]
    if with_deg:
        out_type.append(jax.ShapeDtypeStruct((NP, 8), jnp.float32))
        scratch.append(pltpu.VMEM((CHUNK, 8), jnp.float32))
    scratch.append(pltpu.VMEM_SHARED((NP, HH), jnp.float32))
    if with_deg:
        scratch.append(pltpu.VMEM_SHARED((NP, 8), jnp.float32))
    scratch += [pltpu.SemaphoreType.DMA, pltpu.SemaphoreType.DMA]

    fn = pl.kernel(
        body,
        out_type=tuple(out_type),
        mesh=mesh,
        scratch_types=scratch,
        compiler_params=pltpu.CompilerParams(use_tc_tiling_on_sc=False),
    )
    if with_deg:
        return fn(h2, ei3, ones8, z64, z8)
    return fn(h2, ei3, z64)[0]


def _tc_layer(h2, part, deg8, Wself, Wneigh, b, relu, split_out):
    BN = 1024

    def body(h_ref, p_ref, d_ref, ws_ref, wn_ref, b_ref, o_ref):
        h = jnp.concatenate([h_ref[0], h_ref[1]], axis=1)
        deg = d_ref[:, 0:1]
        inv = 1.0 / jnp.maximum(deg, 1.0)
        mean = jnp.concatenate([p_ref[0], p_ref[1]], axis=1) * inv
        out = (jnp.dot(h, ws_ref[...], preferred_element_type=jnp.float32)
               + jnp.dot(mean, wn_ref[...], preferred_element_type=jnp.float32)
               + b_ref[...])
        if relu:
            out = jnp.maximum(out, 0.0)
        if split_out:
            o_ref[0] = out[:, :HH]
            o_ref[1] = out[:, HH:]
        else:
            o_ref[...] = out

    if split_out:
        out_shape = jax.ShapeDtypeStruct((2, NP, HH), jnp.float32)
        out_specs = pl.BlockSpec((2, BN, HH), lambda i: (0, i, 0))
    else:
        out_shape = jax.ShapeDtypeStruct((NP, H), jnp.float32)
        out_specs = pl.BlockSpec((BN, H), lambda i: (i, 0))

    return pl.pallas_call(
        body,
        grid=(NP // BN,),
        in_specs=[
            pl.BlockSpec((2, BN, HH), lambda i: (0, i, 0)),
            pl.BlockSpec((2, BN, HH), lambda i: (0, i, 0)),
            pl.BlockSpec((BN, 8), lambda i: (i, 0)),
            pl.BlockSpec((H, H), lambda i: (0, 0)),
            pl.BlockSpec((H, H), lambda i: (0, 0)),
            pl.BlockSpec((1, H), lambda i: (0, 0)),
        ],
        out_specs=out_specs,
        out_shape=out_shape,
    )(h2, part, deg8, Wself, Wneigh, b.reshape(1, H))


PG = 4 * NPAIR
GPW = PG // 32
GCH = GPW // 128


def _sc_gather(h, idxT):
    mesh = plsc.VectorSubcoreMesh(core_axis_name="c", subcore_axis_name="s")

    def body(h_hbm, idx_hbm, out_hbm, idx_v,
             rows0, rows1, rows2, rows3, sem_g, sem_w):
        c = lax.axis_index("c")
        s = lax.axis_index("s")
        wid = s * 2 + c
        pltpu.sync_copy(idx_hbm.at[pl.ds(wid * GCH, GCH)], idx_v)
        bufs = (rows0, rows1, rows2, rows3)

        def out_at(j):
            return out_hbm.at[pl.ds(wid * GPW + j * 128, 128)]

        def start_g(j, k):
            pltpu.async_copy(h_hbm.at[idx_v.at[j]], bufs[k], sem_g)

        def stage(j, k, do_wait_w, do_start_g):
            if do_wait_w:
                pltpu.make_async_copy(
                    bufs[(k + 2) % 4], out_at(j - 2), sem_w).wait()
            pltpu.make_async_copy(h_hbm.at[idx_v.at[j]], bufs[k], sem_g).wait()
            pltpu.async_copy(bufs[k], out_at(j), sem_w)
            if do_start_g:
                start_g(j + 2, (k + 2) % 4)

        start_g(0, 0)
        start_g(1, 1)
        stage(0, 0, False, True)
        stage(1, 1, False, True)
        stage(2, 2, True, True)
        stage(3, 3, True, True)

        def group(g, carry):
            j = 4 * g
            stage(j + 0, 0, True, True)
            stage(j + 1, 1, True, True)
            stage(j + 2, 2, True, True)
            stage(j + 3, 3, True, True)
            return carry

        lax.fori_loop(1, GCH // 4 - 1, group, 0)
        j = GCH - 4
        stage(j + 0, 0, True, True)
        stage(j + 1, 1, True, True)
        stage(j + 2, 2, True, False)
        stage(j + 3, 3, True, False)
        pltpu.make_async_copy(bufs[2], out_at(GCH - 2), sem_w).wait()
        pltpu.make_async_copy(bufs[3], out_at(GCH - 1), sem_w).wait()

    fn = pl.kernel(
        body,
        out_type=jax.ShapeDtypeStruct((PG, H), jnp.float32),
        mesh=mesh,
        scratch_types=[
            pltpu.VMEM((GCH, 128), jnp.int32),
            pltpu.VMEM((128, H), jnp.float32),
            pltpu.VMEM((128, H), jnp.float32),
            pltpu.VMEM((128, H), jnp.float32),
            pltpu.VMEM((128, H), jnp.float32),
            pltpu.SemaphoreType.DMA,
            pltpu.SemaphoreType.DMA,
        ],
    )
    return fn(h, idxT)


def _tc_pred(rows, P0w, P0b, P1w, P1b, P2w, P2b):
    BP = 2048
    M = 2 * NPAIR
    OFF = M // BP

    def body(a_ref, b_ref, w0, c0, w1, c1, w2, c2, o_ref):
        z = a_ref[...] * b_ref[...]
        z = jnp.maximum(jnp.dot(z, w0[...], preferred_element_type=jnp.float32) + c0[...], 0.0)
        z = jnp.maximum(jnp.dot(z, w1[...], preferred_element_type=jnp.float32) + c1[...], 0.0)
        logit = jnp.dot(z, w2[...], preferred_element_type=jnp.float32) + c2[...]
        d = logit[:, 1:2] - logit[:, 0:1]
        o_ref[...] = 1.0 / (1.0 + jnp.exp(-d))

    return pl.pallas_call(
        body,
        grid=(M // BP,),
        in_specs=[
            pl.BlockSpec((BP, H), lambda i: (i, 0)),
            pl.BlockSpec((BP, H), lambda i: (i + OFF, 0)),
            pl.BlockSpec((H, H), lambda i: (0, 0)),
            pl.BlockSpec((1, H), lambda i: (0, 0)),
            pl.BlockSpec((H, H), lambda i: (0, 0)),
            pl.BlockSpec((1, H), lambda i: (0, 0)),
            pl.BlockSpec((H, 2), lambda i: (0, 0)),
            pl.BlockSpec((1, 2), lambda i: (0, 0)),
        ],
        out_specs=pl.BlockSpec((BP, 1), lambda i: (i, 0)),
        out_shape=jax.ShapeDtypeStruct((M, 1), jnp.float32),
    )(rows, rows, P0w, P0b.reshape(1, H), P1w, P1b.reshape(1, H), P2w,
      P2b.reshape(1, 2))


def kernel(x, edge_index, pos_src, pos_dst, neg_src, neg_dst,
           Wself0, Wneigh0, b0, Wself1, Wneigh1, b1, Wself2, Wneigh2, b2,
           P0w, P0b, P1w, P1b, P2w, P2b):
    ei3 = edge_index.reshape(2, E // CHUNK, CHUNK)
    ones8 = jnp.ones((CHUNK, 8), jnp.float32)
    z64 = jnp.zeros((RPS, HH), jnp.float32)
    z8 = jnp.zeros((RPS, 8), jnp.float32)

    xp = jnp.pad(x, ((0, NP - N), (0, 0)))
    h2 = jnp.stack([xp[:, :HH], xp[:, HH:]])
    part, deg8 = _sc_agg(h2, ei3, ones8, z64, z8, True)
    h2 = _tc_layer(h2, part, deg8, Wself0, Wneigh0, b0, True, True)
    part = _sc_agg(h2, ei3, ones8, z64, z8, False)
    h2 = _tc_layer(h2, part, deg8, Wself1, Wneigh1, b1, True, True)
    part = _sc_agg(h2, ei3, ones8, z64, z8, False)
    h = _tc_layer(h2, part, deg8, Wself2, Wneigh2, b2, False, False)

    idxT = jnp.concatenate([pos_src, neg_src, pos_dst, neg_dst]).reshape(
        PG // 128, 128)
    rows = _sc_gather(h, idxT)
    scores = _tc_pred(rows, P0w, P0b, P1w, P1b, P2w, P2b)
    scores2 = scores.reshape(2, NPAIR)
    return scores2[0], scores2[1]


# speedup vs baseline: 9.4869x; 1.0089x over previous
import jax
import jax.numpy as jnp
from jax import lax
from jax.experimental import pallas as pl
from jax.experimental.pallas import tpu as pltpu
from jax.experimental.pallas import tpu_sc as plsc

N = 10000
NP = 10240
E = 320000
H = 128
HH = H // 2
NPAIR = 16384

CHUNK = 125
EPS = E // 16
NCH = EPS // CHUNK
RPS = NP // 16


def _sc_agg(h2, ei3, ones8, z64, z8, with_deg):
    mesh = plsc.VectorSubcoreMesh(core_axis_name="c", subcore_axis_name="s")

    def body(*refs):
        if with_deg:
            (h2_hbm, ei_hbm, ones_hbm, z64_hbm, z8_hbm,
             out_agg, out_deg, idx_s, idx_d, rows0, rows1, rows2, rows3,
             ones_v, acc, accd, sem_g, sem_s) = refs
        else:
            (h2_hbm, ei_hbm, z64_hbm,
             out_agg, idx_s, idx_d, rows0, rows1, rows2, rows3, acc,
             sem_g, sem_s) = refs
        c = lax.axis_index("c")
        s = lax.axis_index("s")
        pltpu.sync_copy(z64_hbm, acc.at[pl.ds(s * RPS, RPS)])
        pltpu.sync_copy(ei_hbm.at[0, pl.ds(s * NCH, NCH)], idx_s)
        pltpu.sync_copy(ei_hbm.at[1, pl.ds(s * NCH, NCH)], idx_d)
        if with_deg:
            pltpu.sync_copy(ones_hbm, ones_v)
            pltpu.sync_copy(z8_hbm, accd.at[pl.ds(s * RPS, RPS)])
        plsc.subcore_barrier()

        def run(h_ref, do_deg):
            bufs = (rows0, rows1, rows2, rows3)

            def wait_g(j, k):
                pltpu.make_async_copy(
                    h_ref.at[idx_s.at[j]], bufs[k], sem_g).wait()

            def wait_s(j, k):
                pltpu.make_async_copy(
                    bufs[k], acc.at[idx_d.at[j]], sem_s).wait()

            def start_g(j, k):
                pltpu.async_copy(h_ref.at[idx_s.at[j]], bufs[k], sem_g)

            def start_s(j, k):
                pltpu.async_copy(bufs[k], acc.at[idx_d.at[j]], sem_s,
                                 add=True)

            def stage(j, k, do_wait_s, do_start_g):
                if do_wait_s:
                    wait_s(j - 2, (k + 2) % 4)
                wait_g(j, k)
                start_s(j, k)
                if do_deg is not None:
                    @pl.when(do_deg(j))
                    def _():
                        pltpu.sync_copy(ones_v, accd.at[idx_d.at[j]],
                                        add=True)
                if do_start_g:
                    start_g(j + 2, (k + 2) % 4)

            start_g(0, 0)
            start_g(1, 1)
            stage(0, 0, False, True)
            stage(1, 1, False, True)
            stage(2, 2, True, True)
            stage(3, 3, True, True)

            def group(g, carry):
                j = 4 * g
                stage(j + 0, 0, True, True)
                stage(j + 1, 1, True, True)
                stage(j + 2, 2, True, True)
                stage(j + 3, 3, True, True)
                return carry

            lax.fori_loop(1, NCH // 4 - 1, group, 0)
            j = NCH - 4
            stage(j + 0, 0, True, True)
            stage(j + 1, 1, True, True)
            stage(j + 2, 2, True, False)
            stage(j + 3, 3, True, False)
            wait_s(NCH - 2, 2)
            wait_s(NCH - 1, 3)

        @pl.when(c == 0)
        def _():
            run(h2_hbm.at[0], (lambda j: j < NCH // 2) if with_deg else None)

        @pl.when(c == 1)
        def _():
            run(h2_hbm.at[1], (lambda j: j >= NCH // 2) if with_deg else None)

        plsc.subcore_barrier()
        pltpu.sync_copy(acc.at[pl.ds(s * RPS, RPS)],
                        out_agg.at[c, pl.ds(s * RPS, RPS)])

        if with_deg:
            pltpu.sync_copy(accd.at[pl.ds(s * RPS, RPS)],
                            out_deg.at[c, pl.ds(s * RPS, RPS)])

    out_type = [jax.ShapeDtypeStruct((2, NP, HH), jnp.float32)]
    scratch = [
        pltpu.VMEM((NCH, CHUNK), jnp.int32),
        pltpu.VMEM((NCH, CHUNK), jnp.int32),
        pltpu.VMEM((CHUNK, HH), jnp.float32),
        pltpu.VMEM((CHUNK, HH), jnp.float32),
        pltpu.VMEM((CHUNK, HH), jnp.float32),
        pltpu.VMEM((CHUNK, HH), jnp.float32),
    ]
    if with_deg:
        out_type.append(jax.ShapeDtypeStruct((2, NP, 8), jnp.float32))
        scratch.append(pltpu.VMEM((CHUNK, 8), jnp.float32))
    scratch.append(pltpu.VMEM_SHARED((NP, HH), jnp.float32))
    if with_deg:
        scratch.append(pltpu.VMEM_SHARED((NP, 8), jnp.float32))
    scratch += [pltpu.SemaphoreType.DMA, pltpu.SemaphoreType.DMA]

    fn = pl.kernel(
        body,
        out_type=tuple(out_type),
        mesh=mesh,
        scratch_types=scratch,
        compiler_params=pltpu.CompilerParams(use_tc_tiling_on_sc=False),
    )
    if with_deg:
        return fn(h2, ei3, ones8, z64, z8)
    return fn(h2, ei3, z64)[0]


def _tc_layer(h2, part, deg8, Wself, Wneigh, b, relu, split_out):
    BN = 2048

    def body(h_ref, p_ref, d_ref, ws_ref, wn_ref, b_ref, o_ref):
        h = jnp.concatenate([h_ref[0], h_ref[1]], axis=1)
        deg = d_ref[0, :, 0:1] + d_ref[1, :, 0:1]
        inv = 1.0 / jnp.maximum(deg, 1.0)
        mean = jnp.concatenate([p_ref[0], p_ref[1]], axis=1) * inv
        out = (jnp.dot(h, ws_ref[...], preferred_element_type=jnp.float32)
               + jnp.dot(mean, wn_ref[...], preferred_element_type=jnp.float32)
               + b_ref[...])
        if relu:
            out = jnp.maximum(out, 0.0)
        if split_out:
            o_ref[0] = out[:, :HH]
            o_ref[1] = out[:, HH:]
        else:
            o_ref[...] = out

    if split_out:
        out_shape = jax.ShapeDtypeStruct((2, NP, HH), jnp.float32)
        out_specs = pl.BlockSpec((2, BN, HH), lambda i: (0, i, 0))
    else:
        out_shape = jax.ShapeDtypeStruct((NP, H), jnp.float32)
        out_specs = pl.BlockSpec((BN, H), lambda i: (i, 0))

    return pl.pallas_call(
        body,
        grid=(NP // BN,),
        in_specs=[
            pl.BlockSpec((2, BN, HH), lambda i: (0, i, 0)),
            pl.BlockSpec((2, BN, HH), lambda i: (0, i, 0)),
            pl.BlockSpec((2, BN, 8), lambda i: (0, i, 0)),
            pl.BlockSpec((H, H), lambda i: (0, 0)),
            pl.BlockSpec((H, H), lambda i: (0, 0)),
            pl.BlockSpec((1, H), lambda i: (0, 0)),
        ],
        out_specs=out_specs,
        out_shape=out_shape,
    )(h2, part, deg8, Wself, Wneigh, b.reshape(1, H))


PG = 4 * NPAIR
GPW = PG // 32
GCH = GPW // 128


def _sc_gather(h, idxT):
    mesh = plsc.VectorSubcoreMesh(core_axis_name="c", subcore_axis_name="s")

    def body(h_hbm, idx_hbm, out_hbm, idx_v,
             rows0, rows1, rows2, rows3, sem_g, sem_w):
        c = lax.axis_index("c")
        s = lax.axis_index("s")
        wid = s * 2 + c
        pltpu.sync_copy(idx_hbm.at[pl.ds(wid * GCH, GCH)], idx_v)
        bufs = (rows0, rows1, rows2, rows3)

        def out_at(j):
            return out_hbm.at[pl.ds(wid * GPW + j * 128, 128)]

        def start_g(j, k):
            pltpu.async_copy(h_hbm.at[idx_v.at[j]], bufs[k], sem_g)

        def stage(j, k, do_wait_w, do_start_g):
            if do_wait_w:
                pltpu.make_async_copy(
                    bufs[(k + 2) % 4], out_at(j - 2), sem_w).wait()
            pltpu.make_async_copy(h_hbm.at[idx_v.at[j]], bufs[k], sem_g).wait()
            pltpu.async_copy(bufs[k], out_at(j), sem_w)
            if do_start_g:
                start_g(j + 2, (k + 2) % 4)

        start_g(0, 0)
        start_g(1, 1)
        stage(0, 0, False, True)
        stage(1, 1, False, True)
        stage(2, 2, True, True)
        stage(3, 3, True, True)

        def group(g, carry):
            j = 4 * g
            stage(j + 0, 0, True, True)
            stage(j + 1, 1, True, True)
            stage(j + 2, 2, True, True)
            stage(j + 3, 3, True, True)
            return carry

        lax.fori_loop(1, GCH // 4 - 1, group, 0)
        j = GCH - 4
        stage(j + 0, 0, True, True)
        stage(j + 1, 1, True, True)
        stage(j + 2, 2, True, False)
        stage(j + 3, 3, True, False)
        pltpu.make_async_copy(bufs[2], out_at(GCH - 2), sem_w).wait()
        pltpu.make_async_copy(bufs[3], out_at(GCH - 1), sem_w).wait()

    fn = pl.kernel(
        body,
        out_type=jax.ShapeDtypeStruct((PG, H), jnp.float32),
        mesh=mesh,
        scratch_types=[
            pltpu.VMEM((GCH, 128), jnp.int32),
            pltpu.VMEM((128, H), jnp.float32),
            pltpu.VMEM((128, H), jnp.float32),
            pltpu.VMEM((128, H), jnp.float32),
            pltpu.VMEM((128, H), jnp.float32),
            pltpu.SemaphoreType.DMA,
            pltpu.SemaphoreType.DMA,
        ],
    )
    return fn(h, idxT)


def _tc_pred(rows, P0w, P0b, P1w, P1b, P2w, P2b):
    BP = 4096
    M = 2 * NPAIR
    OFF = M // BP

    def body(a_ref, b_ref, w0, c0, w1, c1, w2, c2, o_ref):
        z = a_ref[...] * b_ref[...]
        z = jnp.maximum(jnp.dot(z, w0[...], preferred_element_type=jnp.float32) + c0[...], 0.0)
        z = jnp.maximum(jnp.dot(z, w1[...], preferred_element_type=jnp.float32) + c1[...], 0.0)
        logit = jnp.dot(z, w2[...], preferred_element_type=jnp.float32) + c2[...]
        d = logit[:, 1:2] - logit[:, 0:1]
        o_ref[...] = 1.0 / (1.0 + jnp.exp(-d))

    return pl.pallas_call(
        body,
        grid=(M // BP,),
        in_specs=[
            pl.BlockSpec((BP, H), lambda i: (i, 0)),
            pl.BlockSpec((BP, H), lambda i: (i + OFF, 0)),
            pl.BlockSpec((H, H), lambda i: (0, 0)),
            pl.BlockSpec((1, H), lambda i: (0, 0)),
            pl.BlockSpec((H, H), lambda i: (0, 0)),
            pl.BlockSpec((1, H), lambda i: (0, 0)),
            pl.BlockSpec((H, 2), lambda i: (0, 0)),
            pl.BlockSpec((1, 2), lambda i: (0, 0)),
        ],
        out_specs=pl.BlockSpec((BP, 1), lambda i: (i, 0)),
        out_shape=jax.ShapeDtypeStruct((M, 1), jnp.float32),
    )(rows, rows, P0w, P0b.reshape(1, H), P1w, P1b.reshape(1, H), P2w,
      P2b.reshape(1, 2))


def kernel(x, edge_index, pos_src, pos_dst, neg_src, neg_dst,
           Wself0, Wneigh0, b0, Wself1, Wneigh1, b1, Wself2, Wneigh2, b2,
           P0w, P0b, P1w, P1b, P2w, P2b):
    ei3 = edge_index.reshape(2, E // CHUNK, CHUNK)
    ones8 = jnp.ones((CHUNK, 8), jnp.float32)
    z64 = jnp.zeros((RPS, HH), jnp.float32)
    z8 = jnp.zeros((RPS, 8), jnp.float32)

    xp = jnp.pad(x, ((0, NP - N), (0, 0)))
    h2 = jnp.stack([xp[:, :HH], xp[:, HH:]])
    part, deg8 = _sc_agg(h2, ei3, ones8, z64, z8, True)
    h2 = _tc_layer(h2, part, deg8, Wself0, Wneigh0, b0, True, True)
    part = _sc_agg(h2, ei3, ones8, z64, z8, False)
    h2 = _tc_layer(h2, part, deg8, Wself1, Wneigh1, b1, True, True)
    part = _sc_agg(h2, ei3, ones8, z64, z8, False)
    h = _tc_layer(h2, part, deg8, Wself2, Wneigh2, b2, False, False)

    idxT = jnp.concatenate([pos_src, neg_src, pos_dst, neg_dst]).reshape(
        PG // 128, 128)
    rows = _sc_gather(h, idxT)
    scores = _tc_pred(rows, P0w, P0b, P1w, P1b, P2w, P2b)
    scores2 = scores.reshape(2, NPAIR)
    return scores2[0], scores2[1]


# speedup vs baseline: 9.7756x; 1.0304x over previous
import jax
import jax.numpy as jnp
from jax import lax
from jax.experimental import pallas as pl
from jax.experimental.pallas import tpu as pltpu
from jax.experimental.pallas import tpu_sc as plsc

N = 10000
NP = 10240
E = 320000
H = 128
HH = H // 2
NPAIR = 16384

CHUNK = 125
EPS = E // 16
NCH = EPS // CHUNK
RPS = NP // 16


def _sc_agg(h2, ei3, ones8, z64, z8, with_deg):
    mesh = plsc.VectorSubcoreMesh(core_axis_name="c", subcore_axis_name="s")

    def body(*refs):
        if with_deg:
            (h2_hbm, ei_hbm, ones_hbm, z64_hbm, z8_hbm,
             out_agg, out_deg, idx_s, idx_d, rows0, rows1, rows2, rows3,
             ones_v, acc, accd, sem_g, sem_s) = refs
        else:
            (h2_hbm, ei_hbm, z64_hbm,
             out_agg, idx_s, idx_d, rows0, rows1, rows2, rows3, acc,
             sem_g, sem_s) = refs
        c = lax.axis_index("c")
        s = lax.axis_index("s")
        pltpu.sync_copy(z64_hbm, acc.at[pl.ds(s * RPS, RPS)])
        pltpu.sync_copy(ei_hbm.at[0, pl.ds(s * NCH, NCH)], idx_s)
        pltpu.sync_copy(ei_hbm.at[1, pl.ds(s * NCH, NCH)], idx_d)
        if with_deg:
            pltpu.sync_copy(ones_hbm, ones_v)
            pltpu.sync_copy(z8_hbm, accd.at[pl.ds(s * RPS, RPS)])
        plsc.subcore_barrier()

        def run(h_ref, do_deg):
            bufs = (rows0, rows1, rows2, rows3)

            def wait_g(j, k):
                pltpu.make_async_copy(
                    h_ref.at[idx_s.at[j]], bufs[k], sem_g).wait()

            def wait_s(j, k):
                pltpu.make_async_copy(
                    bufs[k], acc.at[idx_d.at[j]], sem_s).wait()

            def start_g(j, k):
                pltpu.async_copy(h_ref.at[idx_s.at[j]], bufs[k], sem_g)

            def start_s(j, k):
                pltpu.async_copy(bufs[k], acc.at[idx_d.at[j]], sem_s,
                                 add=True)

            def stage(j, k, do_wait_s, do_start_g):
                if do_wait_s:
                    wait_s(j - 2, (k + 2) % 4)
                wait_g(j, k)
                start_s(j, k)
                if do_deg is not None:
                    @pl.when(do_deg(j))
                    def _():
                        pltpu.sync_copy(ones_v, accd.at[idx_d.at[j]],
                                        add=True)
                if do_start_g:
                    start_g(j + 2, (k + 2) % 4)

            start_g(0, 0)
            start_g(1, 1)
            stage(0, 0, False, True)
            stage(1, 1, False, True)
            stage(2, 2, True, True)
            stage(3, 3, True, True)

            def group(g, carry):
                j = 4 * g
                stage(j + 0, 0, True, True)
                stage(j + 1, 1, True, True)
                stage(j + 2, 2, True, True)
                stage(j + 3, 3, True, True)
                return carry

            lax.fori_loop(1, NCH // 4 - 1, group, 0)
            j = NCH - 4
            stage(j + 0, 0, True, True)
            stage(j + 1, 1, True, True)
            stage(j + 2, 2, True, False)
            stage(j + 3, 3, True, False)
            wait_s(NCH - 2, 2)
            wait_s(NCH - 1, 3)

        @pl.when(c == 0)
        def _():
            run(h2_hbm.at[0], (lambda j: j < NCH // 2) if with_deg else None)

        @pl.when(c == 1)
        def _():
            run(h2_hbm.at[1], (lambda j: j >= NCH // 2) if with_deg else None)

        plsc.subcore_barrier()
        pltpu.sync_copy(acc.at[pl.ds(s * RPS, RPS)],
                        out_agg.at[c, pl.ds(s * RPS, RPS)])

        if with_deg:
            pltpu.sync_copy(accd.at[pl.ds(s * RPS, RPS)],
                            out_deg.at[c, pl.ds(s * RPS, RPS)])

    out_type = [jax.ShapeDtypeStruct((2, NP, HH), jnp.float32)]
    scratch = [
        pltpu.VMEM((NCH, CHUNK), jnp.int32),
        pltpu.VMEM((NCH, CHUNK), jnp.int32),
        pltpu.VMEM((CHUNK, HH), jnp.float32),
        pltpu.VMEM((CHUNK, HH), jnp.float32),
        pltpu.VMEM((CHUNK, HH), jnp.float32),
        pltpu.VMEM((CHUNK, HH), jnp.float32),
    ]
    if with_deg:
        out_type.append(jax.ShapeDtypeStruct((2, NP, 8), jnp.float32))
        scratch.append(pltpu.VMEM((CHUNK, 8), jnp.float32))
    scratch.append(pltpu.VMEM_SHARED((NP, HH), jnp.float32))
    if with_deg:
        scratch.append(pltpu.VMEM_SHARED((NP, 8), jnp.float32))
    scratch += [pltpu.SemaphoreType.DMA, pltpu.SemaphoreType.DMA]

    fn = pl.kernel(
        body,
        out_type=tuple(out_type),
        mesh=mesh,
        scratch_types=scratch,
        compiler_params=pltpu.CompilerParams(use_tc_tiling_on_sc=False),
    )
    if with_deg:
        return fn(h2, ei3, ones8, z64, z8)
    return fn(h2, ei3, z64)[0]


def _tc_layer(h2, part, deg8, Wself, Wneigh, b, relu, split_out):
    BN = 2048

    def body(h_ref, p_ref, d_ref, ws_ref, wn_ref, b_ref, o_ref):
        h = jnp.concatenate([h_ref[0], h_ref[1]], axis=1)
        deg = d_ref[0, :, 0:1] + d_ref[1, :, 0:1]
        inv = 1.0 / jnp.maximum(deg, 1.0)
        mean = jnp.concatenate([p_ref[0], p_ref[1]], axis=1) * inv
        out = (jnp.dot(h, ws_ref[...], preferred_element_type=jnp.float32)
               + jnp.dot(mean, wn_ref[...], preferred_element_type=jnp.float32)
               + b_ref[...])
        if relu:
            out = jnp.maximum(out, 0.0)
        if split_out:
            o_ref[0] = out[:, :HH]
            o_ref[1] = out[:, HH:]
        else:
            o_ref[...] = out

    if split_out:
        out_shape = jax.ShapeDtypeStruct((2, NP, HH), jnp.float32)
        out_specs = pl.BlockSpec((2, BN, HH), lambda i: (0, i, 0))
    else:
        out_shape = jax.ShapeDtypeStruct((NP, H), jnp.float32)
        out_specs = pl.BlockSpec((BN, H), lambda i: (i, 0))

    return pl.pallas_call(
        body,
        grid=(NP // BN,),
        in_specs=[
            pl.BlockSpec((2, BN, HH), lambda i: (0, i, 0)),
            pl.BlockSpec((2, BN, HH), lambda i: (0, i, 0)),
            pl.BlockSpec((2, BN, 8), lambda i: (0, i, 0)),
            pl.BlockSpec((H, H), lambda i: (0, 0)),
            pl.BlockSpec((H, H), lambda i: (0, 0)),
            pl.BlockSpec((1, H), lambda i: (0, 0)),
        ],
        out_specs=out_specs,
        out_shape=out_shape,
    )(h2, part, deg8, Wself, Wneigh, b.reshape(1, H))


PG = 4 * NPAIR
GPW = PG // 32
GCH = GPW // 128


def _sc_gather(h, idxT):
    mesh = plsc.VectorSubcoreMesh(core_axis_name="c", subcore_axis_name="s")

    def body(h_hbm, idx_hbm, out_hbm, idx_v,
             rows0, rows1, rows2, rows3, sem_g, sem_w):
        c = lax.axis_index("c")
        s = lax.axis_index("s")
        wid = s * 2 + c
        pltpu.sync_copy(idx_hbm.at[pl.ds(wid * GCH, GCH)], idx_v)
        bufs = (rows0, rows1, rows2, rows3)

        def out_at(j):
            return out_hbm.at[pl.ds(wid * GPW + j * 128, 128)]

        def start_g(j, k):
            pltpu.async_copy(h_hbm.at[idx_v.at[j]], bufs[k], sem_g)

        def stage(j, k, do_wait_w, do_start_g):
            if do_wait_w:
                pltpu.make_async_copy(
                    bufs[(k + 2) % 4], out_at(j - 2), sem_w).wait()
            pltpu.make_async_copy(h_hbm.at[idx_v.at[j]], bufs[k], sem_g).wait()
            pltpu.async_copy(bufs[k], out_at(j), sem_w)
            if do_start_g:
                start_g(j + 2, (k + 2) % 4)

        start_g(0, 0)
        start_g(1, 1)
        stage(0, 0, False, True)
        stage(1, 1, False, True)
        stage(2, 2, True, True)
        stage(3, 3, True, True)

        def group(g, carry):
            j = 4 * g
            stage(j + 0, 0, True, True)
            stage(j + 1, 1, True, True)
            stage(j + 2, 2, True, True)
            stage(j + 3, 3, True, True)
            return carry

        lax.fori_loop(1, GCH // 4 - 1, group, 0)
        j = GCH - 4
        stage(j + 0, 0, True, True)
        stage(j + 1, 1, True, True)
        stage(j + 2, 2, True, False)
        stage(j + 3, 3, True, False)
        pltpu.make_async_copy(bufs[2], out_at(GCH - 2), sem_w).wait()
        pltpu.make_async_copy(bufs[3], out_at(GCH - 1), sem_w).wait()

    fn = pl.kernel(
        body,
        out_type=jax.ShapeDtypeStruct((PG, H), jnp.float32),
        mesh=mesh,
        scratch_types=[
            pltpu.VMEM((GCH, 128), jnp.int32),
            pltpu.VMEM((128, H), jnp.float32),
            pltpu.VMEM((128, H), jnp.float32),
            pltpu.VMEM((128, H), jnp.float32),
            pltpu.VMEM((128, H), jnp.float32),
            pltpu.SemaphoreType.DMA,
            pltpu.SemaphoreType.DMA,
        ],
    )
    return fn(h, idxT)


def _tc_pred(rows, P0w, P0b, P1w, P1b, P2w, P2b):
    BP = 4096
    M = 2 * NPAIR
    OFF = M // BP

    def body(a_ref, b_ref, w0, c0, w1, c1, w2, c2, o_ref):
        z = a_ref[...] * b_ref[...]
        z = jnp.maximum(jnp.dot(z, w0[...], preferred_element_type=jnp.float32) + c0[...], 0.0)
        z = jnp.maximum(jnp.dot(z, w1[...], preferred_element_type=jnp.float32) + c1[...], 0.0)
        d = lax.dot_general(w2[...], z, (((1,), (1,)), ((), ())),
                            preferred_element_type=jnp.float32) + c2[...]
        o_ref[...] = (1.0 / (1.0 + jnp.exp(-d))).reshape(1, 1, BP)

    w2d = (P2w[:, 1] - P2w[:, 0]).reshape(1, H)
    b2d = (P2b[1] - P2b[0]).reshape(1, 1)
    return pl.pallas_call(
        body,
        grid=(M // BP,),
        in_specs=[
            pl.BlockSpec((BP, H), lambda i: (i, 0)),
            pl.BlockSpec((BP, H), lambda i: (i + OFF, 0)),
            pl.BlockSpec((H, H), lambda i: (0, 0)),
            pl.BlockSpec((1, H), lambda i: (0, 0)),
            pl.BlockSpec((H, H), lambda i: (0, 0)),
            pl.BlockSpec((1, H), lambda i: (0, 0)),
            pl.BlockSpec((1, H), lambda i: (0, 0)),
            pl.BlockSpec((1, 1), lambda i: (0, 0)),
        ],
        out_specs=pl.BlockSpec((1, 1, BP), lambda i: (i, 0, 0)),
        out_shape=jax.ShapeDtypeStruct((M // BP, 1, BP), jnp.float32),
    )(rows, rows, P0w, P0b.reshape(1, H), P1w, P1b.reshape(1, H), w2d, b2d)


def kernel(x, edge_index, pos_src, pos_dst, neg_src, neg_dst,
           Wself0, Wneigh0, b0, Wself1, Wneigh1, b1, Wself2, Wneigh2, b2,
           P0w, P0b, P1w, P1b, P2w, P2b):
    ei3 = edge_index.reshape(2, E // CHUNK, CHUNK)
    ones8 = jnp.ones((CHUNK, 8), jnp.float32)
    z64 = jnp.zeros((RPS, HH), jnp.float32)
    z8 = jnp.zeros((RPS, 8), jnp.float32)

    xp = jnp.pad(x, ((0, NP - N), (0, 0)))
    h2 = jnp.stack([xp[:, :HH], xp[:, HH:]])
    part, deg8 = _sc_agg(h2, ei3, ones8, z64, z8, True)
    h2 = _tc_layer(h2, part, deg8, Wself0, Wneigh0, b0, True, True)
    part = _sc_agg(h2, ei3, ones8, z64, z8, False)
    h2 = _tc_layer(h2, part, deg8, Wself1, Wneigh1, b1, True, True)
    part = _sc_agg(h2, ei3, ones8, z64, z8, False)
    h = _tc_layer(h2, part, deg8, Wself2, Wneigh2, b2, False, False)

    idxT = jnp.concatenate([pos_src, neg_src, pos_dst, neg_dst]).reshape(
        PG // 128, 128)
    rows = _sc_gather(h, idxT)
    scores = _tc_pred(rows, P0w, P0b, P1w, P1b, P2w, P2b)
    half = scores.shape[0] // 2
    return scores[:half].reshape(NPAIR), scores[half:].reshape(NPAIR)
